# Initial kernel scaffold; baseline (speedup 1.0000x reference)
import jax
import jax.numpy as jnp
from jax.experimental import pallas as pl


def kernel(x, edge_index, edge_attr, Wq, bq, Wk, bk, Wv, bv, We, be, Aw, VeRow, projW, projb):
    raise NotImplementedError("write your pallas kernel here")


# speedup vs baseline: 42.7361x; 42.7361x over previous
import functools
import numpy as np
import jax
import jax.numpy as jnp
from jax import lax
from jax.experimental import pallas as pl
from jax.experimental.pallas import tpu as pltpu
from jax.experimental.pallas import tpu_sc as plsc

N = 10000
E = 320000
IN_DIM = 128
H = 8
D = 16
HD = H * D
CLAMP = 5.0

NC = 2
NSUB = 16
NW = NC * NSUB
CH = 128
NCHUNK = E // CH
NP = 10240
NST = NP // NSUB
NPA = NP // 8
NSTA = NPA // NSUB


def _qkv_body(x_ref, w_ref, b_ref, q_ref, k_ref, v_ref):
    out = jnp.dot(x_ref[:], w_ref[:], preferred_element_type=jnp.float32)
    out = out + b_ref[:]
    q_ref[:] = out[:, 0:HD]
    k_ref[:] = out[:, HD:2 * HD]
    v_ref[:] = out[:, 2 * HD:3 * HD]


def _qkv(x, w3, b3):
    bn = 1000
    grid = N // bn
    return pl.pallas_call(
        _qkv_body,
        grid=(grid,),
        in_specs=[
            pl.BlockSpec((bn, IN_DIM), lambda i: (i, 0)),
            pl.BlockSpec((IN_DIM, 3 * HD), lambda i: (0, 0)),
            pl.BlockSpec((1, 3 * HD), lambda i: (0, 0)),
        ],
        out_specs=[
            pl.BlockSpec((bn, HD), lambda i: (i, 0)),
            pl.BlockSpec((bn, HD), lambda i: (i, 0)),
            pl.BlockSpec((bn, HD), lambda i: (i, 0)),
        ],
        out_shape=[jax.ShapeDtypeStruct((N, HD), jnp.float32)] * 3,
    )(x, w3, b3)


def _gather_body(ktab, qtab, vtab, src, dst, kg, qg, vg,
                 sidx, didx, kbuf, qbuf, vbuf, sem):
    c = lax.axis_index("c")
    s = lax.axis_index("s")
    wid = s * NC + c

    def body(t, _):
        g = wid + NW * t
        off = pl.multiple_of(g * CH, CH)
        pltpu.sync_copy(src.at[pl.ds(off, CH)], sidx)
        pltpu.sync_copy(dst.at[pl.ds(off, CH)], didx)
        pltpu.async_copy(ktab.at[sidx], kbuf, sem).wait()
        pltpu.async_copy(qtab.at[didx], qbuf, sem).wait()
        pltpu.async_copy(vtab.at[sidx], vbuf, sem).wait()
        pltpu.sync_copy(kbuf, kg.at[pl.ds(off, CH)])
        pltpu.sync_copy(qbuf, qg.at[pl.ds(off, CH)])
        pltpu.sync_copy(vbuf, vg.at[pl.ds(off, CH)])
        return 0

    trip = (NCHUNK - wid + NW - 1) // NW
    lax.fori_loop(0, trip, body, 0)


def _gather(ktab, qtab, vtab, src, dst):
    mesh = plsc.VectorSubcoreMesh(core_axis_name="c", subcore_axis_name="s")
    f = pl.kernel(
        _gather_body,
        out_type=[jax.ShapeDtypeStruct((E, HD), jnp.float32)] * 3,
        mesh=mesh,
        scratch_types=[
            pltpu.VMEM((CH,), jnp.int32),
            pltpu.VMEM((CH,), jnp.int32),
            pltpu.VMEM((CH, HD), jnp.float32),
            pltpu.VMEM((CH, HD), jnp.float32),
            pltpu.VMEM((CH, HD), jnp.float32),
            pltpu.SemaphoreType.DMA,
        ],
    )
    return f(ktab, qtab, vtab, src, dst)


def _edge_body(ea_ref, kg_ref, qg_ref, vg_ref, mask8_ref, wewt_ref, webt_ref,
               bw_ref, bb_ref, awrep_ref, vblk_ref, pick_ref, place_ref,
               we_ref, p1_ref, p2_ref):
    ea = ea_ref[:]
    kq = kg_ref[:] + qg_ref[:]
    ew = jnp.dot(ea, wewt_ref[:], preferred_element_type=jnp.float32) + bw_ref[:]
    eb = jnp.dot(ea, webt_ref[:], preferred_element_type=jnp.float32) + bb_ref[:]
    sc = kq * ew
    root = jnp.sqrt(jnp.abs(sc))
    et = jnp.where(sc >= 0.0, root, -root) + eb
    we_ref[:] = et
    et2 = jnp.dot(et, vblk_ref[:], preferred_element_type=jnp.float32)
    srep = jnp.dot(et, awrep_ref[:], preferred_element_type=jnp.float32)
    srep = jnp.clip(srep, -CLAMP, CLAMP)
    mrep = jnp.exp(srep)
    p1_ref[:] = mrep * (vg_ref[:] + et2)
    m16 = jnp.dot(mrep, pick_ref[:], preferred_element_type=jnp.float32)
    col = lax.broadcasted_iota(jnp.int32, m16.shape, 1)
    aux16 = m16 + jnp.where(col == H, 1.0, 0.0)
    mask8 = mask8_ref[:]
    acc = jnp.zeros(p1_ref.shape, jnp.float32)
    for j in range(8):
        placed = jnp.dot(aux16, place_ref[j], preferred_element_type=jnp.float32)
        acc = acc + mask8[:, j:j + 1] * placed
    p2_ref[:] = acc


def _edge(edge_attr, kg, qg, vg, mask8, wewt, webt, bw, bb, awrep, vblk,
          pick, place):
    be_blk = 512
    grid = E // be_blk
    full = lambda i: (0, 0)
    blk = lambda i: (i, 0)
    full3 = lambda i: (0, 0, 0)
    return pl.pallas_call(
        _edge_body,
        grid=(grid,),
        in_specs=[
            pl.BlockSpec((be_blk, IN_DIM), blk),
            pl.BlockSpec((be_blk, HD), blk),
            pl.BlockSpec((be_blk, HD), blk),
            pl.BlockSpec((be_blk, HD), blk),
            pl.BlockSpec((be_blk, 8), blk),
            pl.BlockSpec((IN_DIM, HD), full),
            pl.BlockSpec((IN_DIM, HD), full),
            pl.BlockSpec((1, HD), full),
            pl.BlockSpec((1, HD), full),
            pl.BlockSpec((HD, HD), full),
            pl.BlockSpec((HD, HD), full),
            pl.BlockSpec((HD, D), full),
            pl.BlockSpec((8, D, HD), full3),
        ],
        out_specs=[
            pl.BlockSpec((be_blk, HD), blk),
            pl.BlockSpec((be_blk, HD), blk),
            pl.BlockSpec((be_blk, HD), blk),
        ],
        out_shape=[
            jax.ShapeDtypeStruct((E, HD), jnp.float32),
            jax.ShapeDtypeStruct((E, HD), jnp.float32),
            jax.ShapeDtypeStruct((E, HD), jnp.float32),
        ],
    )(edge_attr, kg, qg, vg, mask8, wewt, webt, bw, bb, awrep, vblk, pick,
      place)


def _scatter_body(p1, p2w, dst, idx2, z1, o1, o2,
                  acc1, acc2, idxb, idxb2, b1, b2w):
    c = lax.axis_index("c")
    s = lax.axis_index("s")
    wid = s * NC + c
    row0 = pl.multiple_of(s * NST, NST)
    row2 = pl.multiple_of(s * NSTA, NSTA)
    pltpu.sync_copy(z1, b1)
    pltpu.sync_copy(z1.at[pl.ds(0, NSTA)], b2w.at[pl.ds(0, NSTA)])
    pltpu.sync_copy(b2w.at[pl.ds(0, NSTA)], acc2.at[pl.ds(row2, NSTA)])

    def zcp(r, _):
        ro = pl.multiple_of(row0 + r * CH, CH)
        pltpu.sync_copy(b1, acc1.at[pl.ds(ro, CH)])
        return 0

    lax.fori_loop(0, NST // CH, zcp, 0)
    plsc.subcore_barrier()

    def body(t, _):
        g = wid + NW * t
        off = pl.multiple_of(g * CH, CH)
        pltpu.sync_copy(dst.at[pl.ds(off, CH)], idxb)
        pltpu.sync_copy(idx2.at[pl.ds(off, CH)], idxb2)
        pltpu.sync_copy(p1.at[pl.ds(off, CH)], b1)
        pltpu.sync_copy(p2w.at[pl.ds(off, CH)], b2w)
        pltpu.sync_copy(b1, acc1.at[idxb], add=True)
        pltpu.sync_copy(b2w, acc2.at[idxb2], add=True)
        return 0

    trip = (NCHUNK - wid + NW - 1) // NW
    lax.fori_loop(0, trip, body, 0)
    plsc.subcore_barrier()

    def wcp(r, _):
        ro = pl.multiple_of(row0 + r * CH, CH)
        oo = pl.multiple_of(c * NP + ro, CH)
        pltpu.sync_copy(acc1.at[pl.ds(ro, CH)], b1)
        pltpu.sync_copy(b1, o1.at[pl.ds(oo, CH)])
        return 0

    lax.fori_loop(0, NST // CH, wcp, 0)
    oo2 = pl.multiple_of(c * NPA + row2, NSTA)
    pltpu.sync_copy(acc2.at[pl.ds(row2, NSTA)], b2w.at[pl.ds(0, NSTA)])
    pltpu.sync_copy(b2w.at[pl.ds(0, NSTA)], o2.at[pl.ds(oo2, NSTA)])


def _scatter(p1, p2w, dst, idx2):
    mesh = plsc.VectorSubcoreMesh(core_axis_name="c", subcore_axis_name="s")
    f = pl.kernel(
        _scatter_body,
        out_type=[
            jax.ShapeDtypeStruct((NC * NP, HD), jnp.float32),
            jax.ShapeDtypeStruct((NC * NPA, HD), jnp.float32),
        ],
        mesh=mesh,
        scratch_types=[
            pltpu.VMEM_SHARED((NP, HD), jnp.float32),
            pltpu.VMEM_SHARED((NPA, HD), jnp.float32),
            pltpu.VMEM((CH,), jnp.int32),
            pltpu.VMEM((CH,), jnp.int32),
            pltpu.VMEM((CH, HD), jnp.float32),
            pltpu.VMEM((CH, HD), jnp.float32),
        ],
    )
    z1 = jnp.zeros((CH, HD), jnp.float32)
    o1, o2 = f(p1, p2w, dst, idx2, z1)
    return o1.reshape(NC, NP, HD), o2.reshape(NC, NP, D)


def _node_body(a0_ref, a1_ref, x0_ref, x1_ref, exp816_ref, degrep_ref,
               p0_ref, p1_ref, p2_ref, p3_ref, pb_ref, out_ref):
    m = a0_ref[:] + a1_ref[:]
    aux = x0_ref[:] + x1_ref[:]
    srep = jnp.dot(aux, exp816_ref[:], preferred_element_type=jnp.float32)
    degrep = jnp.dot(aux, degrep_ref[:], preferred_element_type=jnp.float32)
    ld = jnp.log1p(degrep)
    wv = m / (srep + 1e-16)
    acc = jnp.dot(wv, p0_ref[:], preferred_element_type=jnp.float32)
    acc = acc + ld * jnp.dot(wv, p1_ref[:], preferred_element_type=jnp.float32)
    acc = acc + jnp.dot(wv, p2_ref[:], preferred_element_type=jnp.float32) / (1.0 + ld)
    acc = acc + (1.0 + 0.5 * ld) * jnp.dot(wv, p3_ref[:], preferred_element_type=jnp.float32)
    out_ref[:] = acc + pb_ref[:]


def _node(a0, a1, x0, x1, exp816, degrep, p0, p1, p2, p3, pb):
    bn = 1000
    grid = N // bn
    full = lambda i: (0, 0)
    blk = lambda i: (i, 0)
    return pl.pallas_call(
        _node_body,
        grid=(grid,),
        in_specs=[
            pl.BlockSpec((bn, HD), blk),
            pl.BlockSpec((bn, HD), blk),
            pl.BlockSpec((bn, D), blk),
            pl.BlockSpec((bn, D), blk),
            pl.BlockSpec((D, HD), full),
            pl.BlockSpec((D, HD), full),
            pl.BlockSpec((HD, HD), full),
            pl.BlockSpec((HD, HD), full),
            pl.BlockSpec((HD, HD), full),
            pl.BlockSpec((HD, HD), full),
            pl.BlockSpec((1, HD), full),
        ],
        out_specs=pl.BlockSpec((bn, HD), blk),
        out_shape=jax.ShapeDtypeStruct((N, HD), jnp.float32),
    )(a0, a1, x0, x1, exp816, degrep, p0, p1, p2, p3, pb)


_PICK = np.zeros((HD, D), np.float32)
_PICK[np.arange(H) * D, np.arange(H)] = 1.0
_EXP816 = np.zeros((D, HD), np.float32)
for _h in range(H):
    _EXP816[_h, _h * D:(_h + 1) * D] = 1.0
_DEGREP = np.zeros((D, HD), np.float32)
_DEGREP[H, :] = 1.0
_PLACE = np.zeros((8, D, HD), np.float32)
for _j in range(8):
    for _t in range(D):
        _PLACE[_j, _t, _j * D + _t] = 1.0


def _blockdiag(blocks):
    out = jnp.zeros((HD, HD), jnp.float32)
    for h, b in enumerate(blocks):
        out = out.at[h * D:(h + 1) * D, h * D:(h + 1) * D].set(b)
    return out


@jax.jit
def kernel(x, edge_index, edge_attr, Wq, bq, Wk, bk, Wv, bv, We, be, Aw,
           VeRow, projW, projb):
    src = edge_index[0].astype(jnp.int32)
    dst = edge_index[1].astype(jnp.int32)

    w3 = jnp.concatenate([Wq.T, Wk.T, Wv.T], axis=1)
    b3 = jnp.concatenate([bq, bk, bv]).reshape(1, 3 * HD)

    we4 = We.reshape(H, 2 * D, IN_DIM)
    wewt = we4[:, :D, :].reshape(HD, IN_DIM).T
    webt = we4[:, D:, :].reshape(HD, IN_DIM).T
    be2 = be.reshape(H, 2 * D)
    bw = be2[:, :D].reshape(1, HD)
    bb = be2[:, D:].reshape(1, HD)

    aw2 = Aw[:, :, 0]
    awrep = _blockdiag([jnp.outer(aw2[:, h], jnp.ones((D,), jnp.float32))
                        for h in range(H)])
    vblk = _blockdiag([VeRow[:, h, :] for h in range(H)])
    pjt = [_blockdiag([projW[:, j * D:(j + 1) * D].T] * H) for j in range(4)]
    pb = jnp.tile(projb, H).reshape(1, HD)

    q, k, v = _qkv(x, w3, b3)
    kg, qg, vg = _gather(k, q, v, src, dst)
    mask8 = jax.nn.one_hot(dst % 8, 8, dtype=jnp.float32)
    idx2 = dst // 8
    wE, pay1, pay2 = _edge(edge_attr, kg, qg, vg, mask8, wewt, webt, bw, bb,
                           awrep, vblk, jnp.asarray(_PICK), jnp.asarray(_PLACE))
    o1, o2 = _scatter(pay1, pay2, dst, idx2)
    o1 = o1[:, :N]
    o2 = o2[:, :N]
    out = _node(o1[0], o1[1], o2[0], o2[1], jnp.asarray(_EXP816),
                jnp.asarray(_DEGREP), pjt[0], pjt[1], pjt[2], pjt[3], pb)
    return out.reshape(N, H, D), wE


# speedup vs baseline: 50.2640x; 1.1761x over previous
import functools
import numpy as np
import jax
import jax.numpy as jnp
from jax import lax
from jax.experimental import pallas as pl
from jax.experimental.pallas import tpu as pltpu
from jax.experimental.pallas import tpu_sc as plsc

N = 10000
E = 320000
IN_DIM = 128
H = 8
D = 16
HD = H * D
CLAMP = 5.0

NC = 2
NSUB = 16
NW = NC * NSUB
CH = 128
NCHUNK = E // CH
NP = 10240
NST = NP // NSUB
NPA = NP // 8
NSTA = NPA // NSUB


def _qkv_body(x_ref, w_ref, b_ref, q_ref, k_ref, v_ref):
    out = jnp.dot(x_ref[:], w_ref[:], preferred_element_type=jnp.float32)
    out = out + b_ref[:]
    q_ref[:] = out[:, 0:HD]
    k_ref[:] = out[:, HD:2 * HD]
    v_ref[:] = out[:, 2 * HD:3 * HD]


def _qkv(x, w3, b3):
    bn = 1000
    grid = N // bn
    return pl.pallas_call(
        _qkv_body,
        grid=(grid,),
        in_specs=[
            pl.BlockSpec((bn, IN_DIM), lambda i: (i, 0)),
            pl.BlockSpec((IN_DIM, 3 * HD), lambda i: (0, 0)),
            pl.BlockSpec((1, 3 * HD), lambda i: (0, 0)),
        ],
        out_specs=[
            pl.BlockSpec((bn, HD), lambda i: (i, 0)),
            pl.BlockSpec((bn, HD), lambda i: (i, 0)),
            pl.BlockSpec((bn, HD), lambda i: (i, 0)),
        ],
        out_shape=[jax.ShapeDtypeStruct((N, HD), jnp.float32)] * 3,
    )(x, w3, b3)


def _gather_body(ktab, qtab, vtab, src, dst, kg, qg, vg,
                 sidx, didx, kbuf, qbuf, vbuf, sem):
    c = lax.axis_index("c")
    s = lax.axis_index("s")
    wid = s * NC + c

    def body(t, _):
        g = wid + NW * t
        off = pl.multiple_of(g * CH, CH)
        pltpu.sync_copy(src.at[pl.ds(off, CH)], sidx)
        pltpu.sync_copy(dst.at[pl.ds(off, CH)], didx)
        pltpu.async_copy(ktab.at[sidx], kbuf, sem).wait()
        pltpu.async_copy(qtab.at[didx], qbuf, sem).wait()
        pltpu.async_copy(vtab.at[sidx], vbuf, sem).wait()
        pltpu.sync_copy(kbuf, kg.at[pl.ds(off, CH)])
        pltpu.sync_copy(qbuf, qg.at[pl.ds(off, CH)])
        pltpu.sync_copy(vbuf, vg.at[pl.ds(off, CH)])
        return 0

    trip = (NCHUNK - wid + NW - 1) // NW
    lax.fori_loop(0, trip, body, 0)


def _gather(ktab, qtab, vtab, src, dst):
    mesh = plsc.VectorSubcoreMesh(core_axis_name="c", subcore_axis_name="s")
    f = pl.kernel(
        _gather_body,
        out_type=[jax.ShapeDtypeStruct((E, HD), jnp.float32)] * 3,
        mesh=mesh,
        scratch_types=[
            pltpu.VMEM((CH,), jnp.int32),
            pltpu.VMEM((CH,), jnp.int32),
            pltpu.VMEM((CH, HD), jnp.float32),
            pltpu.VMEM((CH, HD), jnp.float32),
            pltpu.VMEM((CH, HD), jnp.float32),
            pltpu.SemaphoreType.DMA,
        ],
    )
    return f(ktab, qtab, vtab, src, dst)


def _edge_body(ea_ref, kg_ref, qg_ref, vg_ref, mask8_ref, wewt_ref, webt_ref,
               bw_ref, bb_ref, awrep_ref, vblk_ref, pick_ref, place_ref,
               we_ref, p1_ref, p2_ref):
    ea = ea_ref[:]
    kq = kg_ref[:] + qg_ref[:]
    ew = jnp.dot(ea, wewt_ref[:], preferred_element_type=jnp.float32) + bw_ref[:]
    eb = jnp.dot(ea, webt_ref[:], preferred_element_type=jnp.float32) + bb_ref[:]
    sc = kq * ew
    root = jnp.sqrt(jnp.abs(sc))
    et = jnp.where(sc >= 0.0, root, -root) + eb
    we_ref[:] = et
    et2 = jnp.dot(et, vblk_ref[:], preferred_element_type=jnp.float32)
    srep = jnp.dot(et, awrep_ref[:], preferred_element_type=jnp.float32)
    srep = jnp.clip(srep, -CLAMP, CLAMP)
    mrep = jnp.exp(srep)
    p1_ref[:] = mrep * (vg_ref[:] + et2)
    m16 = jnp.dot(mrep, pick_ref[:], preferred_element_type=jnp.float32)
    col = lax.broadcasted_iota(jnp.int32, m16.shape, 1)
    aux16 = m16 + jnp.where(col == H, 1.0, 0.0)
    mask8 = mask8_ref[:]
    acc = jnp.zeros(p1_ref.shape, jnp.float32)
    for j in range(8):
        placed = jnp.dot(aux16, place_ref[j], preferred_element_type=jnp.float32)
        acc = acc + mask8[:, j:j + 1] * placed
    p2_ref[:] = acc


def _edge(edge_attr, kg, qg, vg, mask8, wewt, webt, bw, bb, awrep, vblk,
          pick, place):
    be_blk = 2000
    grid = E // be_blk
    full = lambda i: (0, 0)
    blk = lambda i: (i, 0)
    full3 = lambda i: (0, 0, 0)
    return pl.pallas_call(
        _edge_body,
        grid=(grid,),
        in_specs=[
            pl.BlockSpec((be_blk, IN_DIM), blk),
            pl.BlockSpec((be_blk, HD), blk),
            pl.BlockSpec((be_blk, HD), blk),
            pl.BlockSpec((be_blk, HD), blk),
            pl.BlockSpec((be_blk, 8), blk),
            pl.BlockSpec((IN_DIM, HD), full),
            pl.BlockSpec((IN_DIM, HD), full),
            pl.BlockSpec((1, HD), full),
            pl.BlockSpec((1, HD), full),
            pl.BlockSpec((HD, HD), full),
            pl.BlockSpec((HD, HD), full),
            pl.BlockSpec((HD, D), full),
            pl.BlockSpec((8, D, HD), full3),
        ],
        out_specs=[
            pl.BlockSpec((be_blk, HD), blk),
            pl.BlockSpec((be_blk, HD), blk),
            pl.BlockSpec((be_blk, HD), blk),
        ],
        out_shape=[
            jax.ShapeDtypeStruct((E, HD), jnp.float32),
            jax.ShapeDtypeStruct((E, HD), jnp.float32),
            jax.ShapeDtypeStruct((E, HD), jnp.float32),
        ],
    )(edge_attr, kg, qg, vg, mask8, wewt, webt, bw, bb, awrep, vblk, pick,
      place)


def _scatter_body(p1, p2w, dst, idx2, z1, o1, o2,
                  acc1, acc2, idxb, idxb2, b1, b2w):
    c = lax.axis_index("c")
    s = lax.axis_index("s")
    wid = s * NC + c
    row0 = pl.multiple_of(s * NST, NST)
    row2 = pl.multiple_of(s * NSTA, NSTA)
    pltpu.sync_copy(z1, b1)
    pltpu.sync_copy(z1.at[pl.ds(0, NSTA)], b2w.at[pl.ds(0, NSTA)])
    pltpu.sync_copy(b2w.at[pl.ds(0, NSTA)], acc2.at[pl.ds(row2, NSTA)])

    def zcp(r, _):
        ro = pl.multiple_of(row0 + r * CH, CH)
        pltpu.sync_copy(b1, acc1.at[pl.ds(ro, CH)])
        return 0

    lax.fori_loop(0, NST // CH, zcp, 0)
    plsc.subcore_barrier()

    def body(t, _):
        g = wid + NW * t
        off = pl.multiple_of(g * CH, CH)
        pltpu.sync_copy(dst.at[pl.ds(off, CH)], idxb)
        pltpu.sync_copy(idx2.at[pl.ds(off, CH)], idxb2)
        pltpu.sync_copy(p1.at[pl.ds(off, CH)], b1)
        pltpu.sync_copy(p2w.at[pl.ds(off, CH)], b2w)
        pltpu.sync_copy(b1, acc1.at[idxb], add=True)
        pltpu.sync_copy(b2w, acc2.at[idxb2], add=True)
        return 0

    trip = (NCHUNK - wid + NW - 1) // NW
    lax.fori_loop(0, trip, body, 0)
    plsc.subcore_barrier()

    def wcp(r, _):
        ro = pl.multiple_of(row0 + r * CH, CH)
        oo = pl.multiple_of(c * NP + ro, CH)
        pltpu.sync_copy(acc1.at[pl.ds(ro, CH)], b1)
        pltpu.sync_copy(b1, o1.at[pl.ds(oo, CH)])
        return 0

    lax.fori_loop(0, NST // CH, wcp, 0)
    oo2 = pl.multiple_of(c * NPA + row2, NSTA)
    pltpu.sync_copy(acc2.at[pl.ds(row2, NSTA)], b2w.at[pl.ds(0, NSTA)])
    pltpu.sync_copy(b2w.at[pl.ds(0, NSTA)], o2.at[pl.ds(oo2, NSTA)])


def _scatter(p1, p2w, dst, idx2):
    mesh = plsc.VectorSubcoreMesh(core_axis_name="c", subcore_axis_name="s")
    f = pl.kernel(
        _scatter_body,
        out_type=[
            jax.ShapeDtypeStruct((NC * NP, HD), jnp.float32),
            jax.ShapeDtypeStruct((NC * NPA, HD), jnp.float32),
        ],
        mesh=mesh,
        scratch_types=[
            pltpu.VMEM_SHARED((NP, HD), jnp.float32),
            pltpu.VMEM_SHARED((NPA, HD), jnp.float32),
            pltpu.VMEM((CH,), jnp.int32),
            pltpu.VMEM((CH,), jnp.int32),
            pltpu.VMEM((CH, HD), jnp.float32),
            pltpu.VMEM((CH, HD), jnp.float32),
        ],
    )
    z1 = jnp.zeros((CH, HD), jnp.float32)
    o1, o2 = f(p1, p2w, dst, idx2, z1)
    return o1.reshape(NC, NP, HD), o2.reshape(NC, NP, D)


def _node_body(a0_ref, a1_ref, x0_ref, x1_ref, exp816_ref, degrep_ref,
               p0_ref, p1_ref, p2_ref, p3_ref, pb_ref, out_ref):
    m = a0_ref[:] + a1_ref[:]
    aux = x0_ref[:] + x1_ref[:]
    srep = jnp.dot(aux, exp816_ref[:], preferred_element_type=jnp.float32)
    degrep = jnp.dot(aux, degrep_ref[:], preferred_element_type=jnp.float32)
    ld = jnp.log1p(degrep)
    wv = m / (srep + 1e-16)
    acc = jnp.dot(wv, p0_ref[:], preferred_element_type=jnp.float32)
    acc = acc + ld * jnp.dot(wv, p1_ref[:], preferred_element_type=jnp.float32)
    acc = acc + jnp.dot(wv, p2_ref[:], preferred_element_type=jnp.float32) / (1.0 + ld)
    acc = acc + (1.0 + 0.5 * ld) * jnp.dot(wv, p3_ref[:], preferred_element_type=jnp.float32)
    out_ref[:] = acc + pb_ref[:]


def _node(a0, a1, x0, x1, exp816, degrep, p0, p1, p2, p3, pb):
    bn = 1000
    grid = N // bn
    full = lambda i: (0, 0)
    blk = lambda i: (i, 0)
    return pl.pallas_call(
        _node_body,
        grid=(grid,),
        in_specs=[
            pl.BlockSpec((bn, HD), blk),
            pl.BlockSpec((bn, HD), blk),
            pl.BlockSpec((bn, D), blk),
            pl.BlockSpec((bn, D), blk),
            pl.BlockSpec((D, HD), full),
            pl.BlockSpec((D, HD), full),
            pl.BlockSpec((HD, HD), full),
            pl.BlockSpec((HD, HD), full),
            pl.BlockSpec((HD, HD), full),
            pl.BlockSpec((HD, HD), full),
            pl.BlockSpec((1, HD), full),
        ],
        out_specs=pl.BlockSpec((bn, HD), blk),
        out_shape=jax.ShapeDtypeStruct((N, HD), jnp.float32),
    )(a0, a1, x0, x1, exp816, degrep, p0, p1, p2, p3, pb)


_PICK = np.zeros((HD, D), np.float32)
_PICK[np.arange(H) * D, np.arange(H)] = 1.0
_EXP816 = np.zeros((D, HD), np.float32)
for _h in range(H):
    _EXP816[_h, _h * D:(_h + 1) * D] = 1.0
_DEGREP = np.zeros((D, HD), np.float32)
_DEGREP[H, :] = 1.0
_PLACE = np.zeros((8, D, HD), np.float32)
for _j in range(8):
    for _t in range(D):
        _PLACE[_j, _t, _j * D + _t] = 1.0


def _blockdiag(blocks):
    out = jnp.zeros((HD, HD), jnp.float32)
    for h, b in enumerate(blocks):
        out = out.at[h * D:(h + 1) * D, h * D:(h + 1) * D].set(b)
    return out


@jax.jit
def kernel(x, edge_index, edge_attr, Wq, bq, Wk, bk, Wv, bv, We, be, Aw,
           VeRow, projW, projb):
    src = edge_index[0].astype(jnp.int32)
    dst = edge_index[1].astype(jnp.int32)

    w3 = jnp.concatenate([Wq.T, Wk.T, Wv.T], axis=1)
    b3 = jnp.concatenate([bq, bk, bv]).reshape(1, 3 * HD)

    we4 = We.reshape(H, 2 * D, IN_DIM)
    wewt = we4[:, :D, :].reshape(HD, IN_DIM).T
    webt = we4[:, D:, :].reshape(HD, IN_DIM).T
    be2 = be.reshape(H, 2 * D)
    bw = be2[:, :D].reshape(1, HD)
    bb = be2[:, D:].reshape(1, HD)

    aw2 = Aw[:, :, 0]
    awrep = _blockdiag([jnp.outer(aw2[:, h], jnp.ones((D,), jnp.float32))
                        for h in range(H)])
    vblk = _blockdiag([VeRow[:, h, :] for h in range(H)])
    pjt = [_blockdiag([projW[:, j * D:(j + 1) * D].T] * H) for j in range(4)]
    pb = jnp.tile(projb, H).reshape(1, HD)

    q, k, v = _qkv(x, w3, b3)
    kg, qg, vg = _gather(k, q, v, src, dst)
    mask8 = jax.nn.one_hot(dst % 8, 8, dtype=jnp.float32)
    idx2 = dst // 8
    wE, pay1, pay2 = _edge(edge_attr, kg, qg, vg, mask8, wewt, webt, bw, bb,
                           awrep, vblk, jnp.asarray(_PICK), jnp.asarray(_PLACE))
    o1, o2 = _scatter(pay1, pay2, dst, idx2)
    o1 = o1[:, :N]
    o2 = o2[:, :N]
    out = _node(o1[0], o1[1], o2[0], o2[1], jnp.asarray(_EXP816),
                jnp.asarray(_DEGREP), pjt[0], pjt[1], pjt[2], pjt[3], pb)
    return out.reshape(N, H, D), wE


# speedup vs baseline: 60.2375x; 1.1984x over previous
import functools
import numpy as np
import jax
import jax.numpy as jnp
from jax import lax
from jax.experimental import pallas as pl
from jax.experimental.pallas import tpu as pltpu
from jax.experimental.pallas import tpu_sc as plsc

N = 10000
E = 320000
IN_DIM = 128
H = 8
D = 16
HD = H * D
CLAMP = 5.0

NC = 2
NSUB = 16
NW = NC * NSUB
CH = 128
NCHUNK = E // CH
NP = 10240
NST = NP // NSUB
NPA = NP // 8
NSTA = NPA // NSUB


def _qkv_body(x_ref, w_ref, b_ref, q_ref, k_ref, v_ref):
    out = jnp.dot(x_ref[:], w_ref[:], preferred_element_type=jnp.float32)
    out = out + b_ref[:]
    q_ref[:] = out[:, 0:HD]
    k_ref[:] = out[:, HD:2 * HD]
    v_ref[:] = out[:, 2 * HD:3 * HD]


def _qkv(x, w3, b3):
    bn = 1000
    grid = N // bn
    return pl.pallas_call(
        _qkv_body,
        grid=(grid,),
        in_specs=[
            pl.BlockSpec((bn, IN_DIM), lambda i: (i, 0)),
            pl.BlockSpec((IN_DIM, 3 * HD), lambda i: (0, 0)),
            pl.BlockSpec((1, 3 * HD), lambda i: (0, 0)),
        ],
        out_specs=[
            pl.BlockSpec((bn, HD), lambda i: (i, 0)),
            pl.BlockSpec((bn, HD), lambda i: (i, 0)),
            pl.BlockSpec((bn, HD), lambda i: (i, 0)),
        ],
        out_shape=[jax.ShapeDtypeStruct((N, HD), jnp.float32)] * 3,
    )(x, w3, b3)


def _gather_body(ktab, qtab, vtab, src, dst, kg, qg, vg,
                 sidx, didx, kbuf, qbuf, vbuf, sem):
    c = lax.axis_index("c")
    s = lax.axis_index("s")
    wid = s * NC + c

    def body(t, _):
        g = wid + NW * t
        off = pl.multiple_of(g * CH, CH)
        i1 = pltpu.async_copy(src.at[pl.ds(off, CH)], sidx, sem)
        i2 = pltpu.async_copy(dst.at[pl.ds(off, CH)], didx, sem)
        i1.wait()
        i2.wait()
        g1 = pltpu.async_copy(ktab.at[sidx], kbuf, sem)
        g2 = pltpu.async_copy(qtab.at[didx], qbuf, sem)
        g3 = pltpu.async_copy(vtab.at[sidx], vbuf, sem)
        g1.wait()
        g2.wait()
        g3.wait()
        w1 = pltpu.async_copy(kbuf, kg.at[pl.ds(off, CH)], sem)
        w2 = pltpu.async_copy(qbuf, qg.at[pl.ds(off, CH)], sem)
        w3 = pltpu.async_copy(vbuf, vg.at[pl.ds(off, CH)], sem)
        w1.wait()
        w2.wait()
        w3.wait()
        return 0

    trip = (NCHUNK - wid + NW - 1) // NW
    lax.fori_loop(0, trip, body, 0)


def _gather(ktab, qtab, vtab, src, dst):
    mesh = plsc.VectorSubcoreMesh(core_axis_name="c", subcore_axis_name="s")
    f = pl.kernel(
        _gather_body,
        out_type=[jax.ShapeDtypeStruct((E, HD), jnp.float32)] * 3,
        mesh=mesh,
        scratch_types=[
            pltpu.VMEM((CH,), jnp.int32),
            pltpu.VMEM((CH,), jnp.int32),
            pltpu.VMEM((CH, HD), jnp.float32),
            pltpu.VMEM((CH, HD), jnp.float32),
            pltpu.VMEM((CH, HD), jnp.float32),
            pltpu.SemaphoreType.DMA,
        ],
    )
    return f(ktab, qtab, vtab, src, dst)


def _edge_body(ea_ref, kg_ref, qg_ref, vg_ref, mask8_ref, wewt_ref, webt_ref,
               bw_ref, bb_ref, awrep_ref, vblk_ref, pick_ref, place_ref,
               we_ref, p1_ref, p2_ref):
    ea = ea_ref[:]
    kq = kg_ref[:] + qg_ref[:]
    ew = jnp.dot(ea, wewt_ref[:], preferred_element_type=jnp.float32) + bw_ref[:]
    eb = jnp.dot(ea, webt_ref[:], preferred_element_type=jnp.float32) + bb_ref[:]
    sc = kq * ew
    root = jnp.sqrt(jnp.abs(sc))
    et = jnp.where(sc >= 0.0, root, -root) + eb
    we_ref[:] = et
    et2 = jnp.dot(et, vblk_ref[:], preferred_element_type=jnp.float32)
    srep = jnp.dot(et, awrep_ref[:], preferred_element_type=jnp.float32)
    srep = jnp.clip(srep, -CLAMP, CLAMP)
    mrep = jnp.exp(srep)
    p1_ref[:] = mrep * (vg_ref[:] + et2)
    m16 = jnp.dot(mrep, pick_ref[:], preferred_element_type=jnp.float32)
    col = lax.broadcasted_iota(jnp.int32, m16.shape, 1)
    aux16 = m16 + jnp.where(col == H, 1.0, 0.0)
    mask8 = mask8_ref[:]
    acc = jnp.zeros(p1_ref.shape, jnp.float32)
    for j in range(8):
        placed = jnp.dot(aux16, place_ref[j], preferred_element_type=jnp.float32)
        acc = acc + mask8[:, j:j + 1] * placed
    p2_ref[:] = acc


def _edge(edge_attr, kg, qg, vg, mask8, wewt, webt, bw, bb, awrep, vblk,
          pick, place):
    be_blk = 2000
    grid = E // be_blk
    full = lambda i: (0, 0)
    blk = lambda i: (i, 0)
    full3 = lambda i: (0, 0, 0)
    return pl.pallas_call(
        _edge_body,
        grid=(grid,),
        in_specs=[
            pl.BlockSpec((be_blk, IN_DIM), blk),
            pl.BlockSpec((be_blk, HD), blk),
            pl.BlockSpec((be_blk, HD), blk),
            pl.BlockSpec((be_blk, HD), blk),
            pl.BlockSpec((be_blk, 8), blk),
            pl.BlockSpec((IN_DIM, HD), full),
            pl.BlockSpec((IN_DIM, HD), full),
            pl.BlockSpec((1, HD), full),
            pl.BlockSpec((1, HD), full),
            pl.BlockSpec((HD, HD), full),
            pl.BlockSpec((HD, HD), full),
            pl.BlockSpec((HD, D), full),
            pl.BlockSpec((8, D, HD), full3),
        ],
        out_specs=[
            pl.BlockSpec((be_blk, HD), blk),
            pl.BlockSpec((be_blk, HD), blk),
            pl.BlockSpec((be_blk, HD), blk),
        ],
        out_shape=[
            jax.ShapeDtypeStruct((E, HD), jnp.float32),
            jax.ShapeDtypeStruct((E, HD), jnp.float32),
            jax.ShapeDtypeStruct((E, HD), jnp.float32),
        ],
    )(edge_attr, kg, qg, vg, mask8, wewt, webt, bw, bb, awrep, vblk, pick,
      place)


def _scatter_body(p1, p2w, dst, idx2, z1, o1, o2,
                  acc1, acc2, idxb, idxb2, b1, b2w, sem):
    c = lax.axis_index("c")
    s = lax.axis_index("s")
    wid = s * NC + c
    row0 = pl.multiple_of(s * NST, NST)
    row2 = pl.multiple_of(s * NSTA, NSTA)
    pltpu.sync_copy(z1, b1)
    pltpu.sync_copy(z1.at[pl.ds(0, NSTA)], b2w.at[pl.ds(0, NSTA)])
    pltpu.sync_copy(b2w.at[pl.ds(0, NSTA)], acc2.at[pl.ds(row2, NSTA)])

    def zcp(r, _):
        ro = pl.multiple_of(row0 + r * CH, CH)
        pltpu.sync_copy(b1, acc1.at[pl.ds(ro, CH)])
        return 0

    lax.fori_loop(0, NST // CH, zcp, 0)
    plsc.subcore_barrier()

    def body(t, _):
        g = wid + NW * t
        off = pl.multiple_of(g * CH, CH)
        l1 = pltpu.async_copy(dst.at[pl.ds(off, CH)], idxb, sem)
        l2 = pltpu.async_copy(idx2.at[pl.ds(off, CH)], idxb2, sem)
        l3 = pltpu.async_copy(p1.at[pl.ds(off, CH)], b1, sem)
        l4 = pltpu.async_copy(p2w.at[pl.ds(off, CH)], b2w, sem)
        l1.wait()
        l2.wait()
        l3.wait()
        l4.wait()
        s1 = pltpu.async_copy(b1, acc1.at[idxb], sem, add=True)
        s2 = pltpu.async_copy(b2w, acc2.at[idxb2], sem, add=True)
        s1.wait()
        s2.wait()
        return 0

    trip = (NCHUNK - wid + NW - 1) // NW
    lax.fori_loop(0, trip, body, 0)
    plsc.subcore_barrier()

    def wcp(r, _):
        ro = pl.multiple_of(row0 + r * CH, CH)
        oo = pl.multiple_of(c * NP + ro, CH)
        pltpu.sync_copy(acc1.at[pl.ds(ro, CH)], b1)
        pltpu.sync_copy(b1, o1.at[pl.ds(oo, CH)])
        return 0

    lax.fori_loop(0, NST // CH, wcp, 0)
    oo2 = pl.multiple_of(c * NPA + row2, NSTA)
    pltpu.sync_copy(acc2.at[pl.ds(row2, NSTA)], b2w.at[pl.ds(0, NSTA)])
    pltpu.sync_copy(b2w.at[pl.ds(0, NSTA)], o2.at[pl.ds(oo2, NSTA)])


def _scatter(p1, p2w, dst, idx2):
    mesh = plsc.VectorSubcoreMesh(core_axis_name="c", subcore_axis_name="s")
    f = pl.kernel(
        _scatter_body,
        out_type=[
            jax.ShapeDtypeStruct((NC * NP, HD), jnp.float32),
            jax.ShapeDtypeStruct((NC * NPA, HD), jnp.float32),
        ],
        mesh=mesh,
        scratch_types=[
            pltpu.VMEM_SHARED((NP, HD), jnp.float32),
            pltpu.VMEM_SHARED((NPA, HD), jnp.float32),
            pltpu.VMEM((CH,), jnp.int32),
            pltpu.VMEM((CH,), jnp.int32),
            pltpu.VMEM((CH, HD), jnp.float32),
            pltpu.VMEM((CH, HD), jnp.float32),
            pltpu.SemaphoreType.DMA,
        ],
    )
    z1 = jnp.zeros((CH, HD), jnp.float32)
    o1, o2 = f(p1, p2w, dst, idx2, z1)
    return o1.reshape(NC, NP, HD), o2.reshape(NC, NP, D)


def _node_body(a0_ref, a1_ref, x0_ref, x1_ref, exp816_ref, degrep_ref,
               p0_ref, p1_ref, p2_ref, p3_ref, pb_ref, out_ref):
    m = a0_ref[:] + a1_ref[:]
    aux = x0_ref[:] + x1_ref[:]
    srep = jnp.dot(aux, exp816_ref[:], preferred_element_type=jnp.float32)
    degrep = jnp.dot(aux, degrep_ref[:], preferred_element_type=jnp.float32)
    ld = jnp.log1p(degrep)
    wv = m / (srep + 1e-16)
    acc = jnp.dot(wv, p0_ref[:], preferred_element_type=jnp.float32)
    acc = acc + ld * jnp.dot(wv, p1_ref[:], preferred_element_type=jnp.float32)
    acc = acc + jnp.dot(wv, p2_ref[:], preferred_element_type=jnp.float32) / (1.0 + ld)
    acc = acc + (1.0 + 0.5 * ld) * jnp.dot(wv, p3_ref[:], preferred_element_type=jnp.float32)
    out_ref[:] = acc + pb_ref[:]


def _node(a0, a1, x0, x1, exp816, degrep, p0, p1, p2, p3, pb):
    bn = 1000
    grid = N // bn
    full = lambda i: (0, 0)
    blk = lambda i: (i, 0)
    return pl.pallas_call(
        _node_body,
        grid=(grid,),
        in_specs=[
            pl.BlockSpec((bn, HD), blk),
            pl.BlockSpec((bn, HD), blk),
            pl.BlockSpec((bn, D), blk),
            pl.BlockSpec((bn, D), blk),
            pl.BlockSpec((D, HD), full),
            pl.BlockSpec((D, HD), full),
            pl.BlockSpec((HD, HD), full),
            pl.BlockSpec((HD, HD), full),
            pl.BlockSpec((HD, HD), full),
            pl.BlockSpec((HD, HD), full),
            pl.BlockSpec((1, HD), full),
        ],
        out_specs=pl.BlockSpec((bn, HD), blk),
        out_shape=jax.ShapeDtypeStruct((N, HD), jnp.float32),
    )(a0, a1, x0, x1, exp816, degrep, p0, p1, p2, p3, pb)


_PICK = np.zeros((HD, D), np.float32)
_PICK[np.arange(H) * D, np.arange(H)] = 1.0
_EXP816 = np.zeros((D, HD), np.float32)
for _h in range(H):
    _EXP816[_h, _h * D:(_h + 1) * D] = 1.0
_DEGREP = np.zeros((D, HD), np.float32)
_DEGREP[H, :] = 1.0
_PLACE = np.zeros((8, D, HD), np.float32)
for _j in range(8):
    for _t in range(D):
        _PLACE[_j, _t, _j * D + _t] = 1.0


def _blockdiag(blocks):
    out = jnp.zeros((HD, HD), jnp.float32)
    for h, b in enumerate(blocks):
        out = out.at[h * D:(h + 1) * D, h * D:(h + 1) * D].set(b)
    return out


@jax.jit
def kernel(x, edge_index, edge_attr, Wq, bq, Wk, bk, Wv, bv, We, be, Aw,
           VeRow, projW, projb):
    src = edge_index[0].astype(jnp.int32)
    dst = edge_index[1].astype(jnp.int32)

    w3 = jnp.concatenate([Wq.T, Wk.T, Wv.T], axis=1)
    b3 = jnp.concatenate([bq, bk, bv]).reshape(1, 3 * HD)

    we4 = We.reshape(H, 2 * D, IN_DIM)
    wewt = we4[:, :D, :].reshape(HD, IN_DIM).T
    webt = we4[:, D:, :].reshape(HD, IN_DIM).T
    be2 = be.reshape(H, 2 * D)
    bw = be2[:, :D].reshape(1, HD)
    bb = be2[:, D:].reshape(1, HD)

    aw2 = Aw[:, :, 0]
    awrep = _blockdiag([jnp.outer(aw2[:, h], jnp.ones((D,), jnp.float32))
                        for h in range(H)])
    vblk = _blockdiag([VeRow[:, h, :] for h in range(H)])
    pjt = [_blockdiag([projW[:, j * D:(j + 1) * D].T] * H) for j in range(4)]
    pb = jnp.tile(projb, H).reshape(1, HD)

    q, k, v = _qkv(x, w3, b3)
    kg, qg, vg = _gather(k, q, v, src, dst)
    mask8 = jax.nn.one_hot(dst % 8, 8, dtype=jnp.float32)
    idx2 = dst // 8
    wE, pay1, pay2 = _edge(edge_attr, kg, qg, vg, mask8, wewt, webt, bw, bb,
                           awrep, vblk, jnp.asarray(_PICK), jnp.asarray(_PLACE))
    o1, o2 = _scatter(pay1, pay2, dst, idx2)
    o1 = o1[:, :N]
    o2 = o2[:, :N]
    out = _node(o1[0], o1[1], o2[0], o2[1], jnp.asarray(_EXP816),
                jnp.asarray(_DEGREP), pjt[0], pjt[1], pjt[2], pjt[3], pb)
    return out.reshape(N, H, D), wE


# speedup vs baseline: 64.3505x; 1.0683x over previous
import functools
import numpy as np
import jax
import jax.numpy as jnp
from jax import lax
from jax.experimental import pallas as pl
from jax.experimental.pallas import tpu as pltpu
from jax.experimental.pallas import tpu_sc as plsc

N = 10000
E = 320000
IN_DIM = 128
H = 8
D = 16
HD = H * D
CLAMP = 5.0

NC = 2
NSUB = 16
NW = NC * NSUB
CH = 128
NCHUNK = E // CH
NP = 10240
NST = NP // NSUB
NPA = NP // 8
NSTA = NPA // NSUB


def _qkv_body(x_ref, w_ref, b_ref, q_ref, k_ref, v_ref):
    out = jnp.dot(x_ref[:], w_ref[:], preferred_element_type=jnp.float32)
    out = out + b_ref[:]
    q_ref[:] = out[:, 0:HD]
    k_ref[:] = out[:, HD:2 * HD]
    v_ref[:] = out[:, 2 * HD:3 * HD]


def _qkv(x, w3, b3):
    bn = 1000
    grid = N // bn
    return pl.pallas_call(
        _qkv_body,
        grid=(grid,),
        in_specs=[
            pl.BlockSpec((bn, IN_DIM), lambda i: (i, 0)),
            pl.BlockSpec((IN_DIM, 3 * HD), lambda i: (0, 0)),
            pl.BlockSpec((1, 3 * HD), lambda i: (0, 0)),
        ],
        out_specs=[
            pl.BlockSpec((bn, HD), lambda i: (i, 0)),
            pl.BlockSpec((bn, HD), lambda i: (i, 0)),
            pl.BlockSpec((bn, HD), lambda i: (i, 0)),
        ],
        out_shape=[jax.ShapeDtypeStruct((N, HD), jnp.float32)] * 3,
    )(x, w3, b3)


def _gather_body(ktab, qtab, vtab, src, dst, kg, qg, vg,
                 sidx, didx, kbuf, qbuf, vbuf, sem):
    c = lax.axis_index("c")
    s = lax.axis_index("s")
    wid = s * NC + c

    def body(t, _):
        g = wid + NW * t
        off = pl.multiple_of(g * CH, CH)
        i1 = pltpu.async_copy(src.at[pl.ds(off, CH)], sidx, sem)
        i2 = pltpu.async_copy(dst.at[pl.ds(off, CH)], didx, sem)
        i1.wait()
        i2.wait()
        g1 = pltpu.async_copy(ktab.at[sidx], kbuf, sem)
        g2 = pltpu.async_copy(qtab.at[didx], qbuf, sem)
        g3 = pltpu.async_copy(vtab.at[sidx], vbuf, sem)
        g1.wait()
        g2.wait()
        g3.wait()
        w1 = pltpu.async_copy(kbuf, kg.at[pl.ds(off, CH)], sem)
        w2 = pltpu.async_copy(qbuf, qg.at[pl.ds(off, CH)], sem)
        w3 = pltpu.async_copy(vbuf, vg.at[pl.ds(off, CH)], sem)
        w1.wait()
        w2.wait()
        w3.wait()
        return 0

    trip = (NCHUNK - wid + NW - 1) // NW
    lax.fori_loop(0, trip, body, 0)


def _gather(ktab, qtab, vtab, src, dst):
    mesh = plsc.VectorSubcoreMesh(core_axis_name="c", subcore_axis_name="s")
    f = pl.kernel(
        _gather_body,
        out_type=[jax.ShapeDtypeStruct((E, HD), jnp.float32)] * 3,
        mesh=mesh,
        scratch_types=[
            pltpu.VMEM((CH,), jnp.int32),
            pltpu.VMEM((CH,), jnp.int32),
            pltpu.VMEM((CH, HD), jnp.float32),
            pltpu.VMEM((CH, HD), jnp.float32),
            pltpu.VMEM((CH, HD), jnp.float32),
            pltpu.SemaphoreType.DMA,
        ],
    )
    return f(ktab, qtab, vtab, src, dst)


def _edge_body(ea_ref, kg_ref, qg_ref, vg_ref, wewt_ref, webt_ref,
               bw_ref, bb_ref, awrep_ref, vblk_ref,
               we_ref, p1_ref, p2_ref):
    ea = ea_ref[:]
    kq = kg_ref[:] + qg_ref[:]
    ew = jnp.dot(ea, wewt_ref[:], preferred_element_type=jnp.float32) + bw_ref[:]
    eb = jnp.dot(ea, webt_ref[:], preferred_element_type=jnp.float32) + bb_ref[:]
    sc = kq * ew
    root = jnp.sqrt(jnp.abs(sc))
    et = jnp.where(sc >= 0.0, root, -root) + eb
    we_ref[:] = et
    et2 = jnp.dot(et, vblk_ref[:], preferred_element_type=jnp.float32)
    srep = jnp.dot(et, awrep_ref[:], preferred_element_type=jnp.float32)
    srep = jnp.clip(srep, -CLAMP, CLAMP)
    mrep = jnp.exp(srep)
    p1_ref[:] = mrep * (vg_ref[:] + et2)
    col = lax.broadcasted_iota(jnp.int32, mrep.shape, 1)
    p2_ref[:] = jnp.where(col == 1, 1.0, mrep)


def _edge(edge_attr, kg, qg, vg, wewt, webt, bw, bb, awrep, vblk):
    be_blk = 2000
    grid = E // be_blk
    full = lambda i: (0, 0)
    blk = lambda i: (i, 0)
    return pl.pallas_call(
        _edge_body,
        grid=(grid,),
        in_specs=[
            pl.BlockSpec((be_blk, IN_DIM), blk),
            pl.BlockSpec((be_blk, HD), blk),
            pl.BlockSpec((be_blk, HD), blk),
            pl.BlockSpec((be_blk, HD), blk),
            pl.BlockSpec((IN_DIM, HD), full),
            pl.BlockSpec((IN_DIM, HD), full),
            pl.BlockSpec((1, HD), full),
            pl.BlockSpec((1, HD), full),
            pl.BlockSpec((HD, HD), full),
            pl.BlockSpec((HD, HD), full),
        ],
        out_specs=[
            pl.BlockSpec((be_blk, HD), blk),
            pl.BlockSpec((be_blk, HD), blk),
            pl.BlockSpec((be_blk, HD), blk),
        ],
        out_shape=[
            jax.ShapeDtypeStruct((E, HD), jnp.float32),
            jax.ShapeDtypeStruct((E, HD), jnp.float32),
            jax.ShapeDtypeStruct((E, HD), jnp.float32),
        ],
    )(edge_attr, kg, qg, vg, wewt, webt, bw, bb, awrep, vblk)


def _scatter_body(p1, p2, dst, z1, o1, o2, acc, idxb, b1, sem):
    c = lax.axis_index("c")
    s = lax.axis_index("s")
    row0 = pl.multiple_of(s * NST, NST)
    pltpu.sync_copy(z1, b1)

    def zcp(r, _):
        ro = pl.multiple_of(row0 + r * CH, CH)
        pltpu.sync_copy(b1, acc.at[pl.ds(ro, CH)])
        return 0

    lax.fori_loop(0, NST // CH, zcp, 0)
    plsc.subcore_barrier()

    def body(t, _):
        g = s + NSUB * t
        off = pl.multiple_of(g * CH, CH)
        l1 = pltpu.async_copy(dst.at[pl.ds(off, CH)], idxb, sem)
        l1.wait()

        @pl.when(c == 0)
        def _():
            pltpu.async_copy(p1.at[pl.ds(off, CH)], b1, sem).wait()

        @pl.when(c == 1)
        def _():
            pltpu.async_copy(p2.at[pl.ds(off, CH)], b1, sem).wait()

        pltpu.sync_copy(b1, acc.at[idxb], add=True)
        return 0

    trip = (NCHUNK - s + NSUB - 1) // NSUB
    lax.fori_loop(0, trip, body, 0)
    plsc.subcore_barrier()

    def wcp(r, _):
        ro = pl.multiple_of(row0 + r * CH, CH)
        pltpu.sync_copy(acc.at[pl.ds(ro, CH)], b1)

        @pl.when(c == 0)
        def _():
            pltpu.async_copy(b1, o1.at[pl.ds(ro, CH)], sem).wait()

        @pl.when(c == 1)
        def _():
            pltpu.async_copy(b1, o2.at[pl.ds(ro, CH)], sem).wait()

        return 0

    lax.fori_loop(0, NST // CH, wcp, 0)


def _scatter(p1, p2, dst):
    mesh = plsc.VectorSubcoreMesh(core_axis_name="c", subcore_axis_name="s")
    f = pl.kernel(
        _scatter_body,
        out_type=[
            jax.ShapeDtypeStruct((NP, HD), jnp.float32),
            jax.ShapeDtypeStruct((NP, HD), jnp.float32),
        ],
        mesh=mesh,
        scratch_types=[
            pltpu.VMEM_SHARED((NP, HD), jnp.float32),
            pltpu.VMEM((CH,), jnp.int32),
            pltpu.VMEM((CH, HD), jnp.float32),
            pltpu.SemaphoreType.DMA,
        ],
    )
    z1 = jnp.zeros((CH, HD), jnp.float32)
    return f(p1, p2, dst, z1)


def _node_body(m_ref, aux_ref, sel_ref, degsel_ref,
               p0_ref, p1_ref, p2_ref, p3_ref, pb_ref, out_ref):
    m = m_ref[:]
    aux = aux_ref[:]
    srep = jnp.dot(aux, sel_ref[:], preferred_element_type=jnp.float32)
    degrep = jnp.dot(aux, degsel_ref[:], preferred_element_type=jnp.float32)
    ld = jnp.log1p(degrep)
    wv = m / (srep + 1e-16)
    acc = jnp.dot(wv, p0_ref[:], preferred_element_type=jnp.float32)
    acc = acc + ld * jnp.dot(wv, p1_ref[:], preferred_element_type=jnp.float32)
    acc = acc + jnp.dot(wv, p2_ref[:], preferred_element_type=jnp.float32) / (1.0 + ld)
    acc = acc + (1.0 + 0.5 * ld) * jnp.dot(wv, p3_ref[:], preferred_element_type=jnp.float32)
    out_ref[:] = acc + pb_ref[:]


def _node(m, aux, sel, degsel, p0, p1, p2, p3, pb):
    bn = 1000
    grid = N // bn
    full = lambda i: (0, 0)
    blk = lambda i: (i, 0)
    return pl.pallas_call(
        _node_body,
        grid=(grid,),
        in_specs=[
            pl.BlockSpec((bn, HD), blk),
            pl.BlockSpec((bn, HD), blk),
            pl.BlockSpec((HD, HD), full),
            pl.BlockSpec((HD, HD), full),
            pl.BlockSpec((HD, HD), full),
            pl.BlockSpec((HD, HD), full),
            pl.BlockSpec((HD, HD), full),
            pl.BlockSpec((HD, HD), full),
            pl.BlockSpec((1, HD), full),
        ],
        out_specs=pl.BlockSpec((bn, HD), blk),
        out_shape=jax.ShapeDtypeStruct((N, HD), jnp.float32),
    )(m, aux, sel, degsel, p0, p1, p2, p3, pb)


_SEL1 = np.zeros((HD, HD), np.float32)
for _h in range(H):
    _SEL1[_h * D, _h * D:(_h + 1) * D] = 1.0
_DEGSEL = np.zeros((HD, HD), np.float32)
_DEGSEL[1, :] = 1.0


def _blockdiag(blocks):
    out = jnp.zeros((HD, HD), jnp.float32)
    for h, b in enumerate(blocks):
        out = out.at[h * D:(h + 1) * D, h * D:(h + 1) * D].set(b)
    return out


@jax.jit
def kernel(x, edge_index, edge_attr, Wq, bq, Wk, bk, Wv, bv, We, be, Aw,
           VeRow, projW, projb):
    src = edge_index[0].astype(jnp.int32)
    dst = edge_index[1].astype(jnp.int32)

    w3 = jnp.concatenate([Wq.T, Wk.T, Wv.T], axis=1)
    b3 = jnp.concatenate([bq, bk, bv]).reshape(1, 3 * HD)

    we4 = We.reshape(H, 2 * D, IN_DIM)
    wewt = we4[:, :D, :].reshape(HD, IN_DIM).T
    webt = we4[:, D:, :].reshape(HD, IN_DIM).T
    be2 = be.reshape(H, 2 * D)
    bw = be2[:, :D].reshape(1, HD)
    bb = be2[:, D:].reshape(1, HD)

    aw2 = Aw[:, :, 0]
    awrep = _blockdiag([jnp.outer(aw2[:, h], jnp.ones((D,), jnp.float32))
                        for h in range(H)])
    vblk = _blockdiag([VeRow[:, h, :] for h in range(H)])
    pjt = [_blockdiag([projW[:, j * D:(j + 1) * D].T] * H) for j in range(4)]
    pb = jnp.tile(projb, H).reshape(1, HD)

    q, k, v = _qkv(x, w3, b3)
    kg, qg, vg = _gather(k, q, v, src, dst)
    wE, pay1, pay2 = _edge(edge_attr, kg, qg, vg, wewt, webt, bw, bb,
                           awrep, vblk)
    o1, o2 = _scatter(pay1, pay2, dst)
    out = _node(o1[:N], o2[:N], jnp.asarray(_SEL1), jnp.asarray(_DEGSEL),
                pjt[0], pjt[1], pjt[2], pjt[3], pb)
    return out.reshape(N, H, D), wE


# speedup vs baseline: 65.8993x; 1.0241x over previous
import functools
import numpy as np
import jax
import jax.numpy as jnp
from jax import lax
from jax.experimental import pallas as pl
from jax.experimental.pallas import tpu as pltpu
from jax.experimental.pallas import tpu_sc as plsc

N = 10000
E = 320000
IN_DIM = 128
H = 8
D = 16
HD = H * D
CLAMP = 5.0

NC = 2
NSUB = 16
NW = NC * NSUB
CH = 128
NCHUNK = E // CH
NP = 10240
NST = NP // NSUB
NPA = NP // 8
NSTA = NPA // NSUB


def _qkv_body(x_ref, w_ref, b_ref, q_ref, k_ref, v_ref):
    out = jnp.dot(x_ref[:], w_ref[:], preferred_element_type=jnp.float32)
    out = out + b_ref[:]
    q_ref[:] = out[:, 0:HD]
    k_ref[:] = out[:, HD:2 * HD]
    v_ref[:] = out[:, 2 * HD:3 * HD]


def _qkv(x, w3, b3):
    bn = 2000
    grid = N // bn
    return pl.pallas_call(
        _qkv_body,
        grid=(grid,),
        in_specs=[
            pl.BlockSpec((bn, IN_DIM), lambda i: (i, 0)),
            pl.BlockSpec((IN_DIM, 3 * HD), lambda i: (0, 0)),
            pl.BlockSpec((1, 3 * HD), lambda i: (0, 0)),
        ],
        out_specs=[
            pl.BlockSpec((bn, HD), lambda i: (i, 0)),
            pl.BlockSpec((bn, HD), lambda i: (i, 0)),
            pl.BlockSpec((bn, HD), lambda i: (i, 0)),
        ],
        out_shape=[jax.ShapeDtypeStruct((N, HD), jnp.float32)] * 3,
    )(x, w3, b3)


def _gather_body(ktab, qtab, vtab, src, dst, kg, qg, vg,
                 sidx, didx, kbuf, qbuf, vbuf, sem):
    c = lax.axis_index("c")
    s = lax.axis_index("s")
    wid = s * NC + c

    def body(t, _):
        g = wid + NW * t
        off = pl.multiple_of(g * CH, CH)
        i1 = pltpu.async_copy(src.at[pl.ds(off, CH)], sidx, sem)
        i2 = pltpu.async_copy(dst.at[pl.ds(off, CH)], didx, sem)
        i1.wait()
        i2.wait()
        g1 = pltpu.async_copy(ktab.at[sidx], kbuf, sem)
        g2 = pltpu.async_copy(qtab.at[didx], qbuf, sem)
        g3 = pltpu.async_copy(vtab.at[sidx], vbuf, sem)
        g1.wait()
        g2.wait()
        g3.wait()
        w1 = pltpu.async_copy(kbuf, kg.at[pl.ds(off, CH)], sem)
        w2 = pltpu.async_copy(qbuf, qg.at[pl.ds(off, CH)], sem)
        w3 = pltpu.async_copy(vbuf, vg.at[pl.ds(off, CH)], sem)
        w1.wait()
        w2.wait()
        w3.wait()
        return 0

    trip = (NCHUNK - wid + NW - 1) // NW
    lax.fori_loop(0, trip, body, 0)


def _gather(ktab, qtab, vtab, src, dst):
    mesh = plsc.VectorSubcoreMesh(core_axis_name="c", subcore_axis_name="s")
    f = pl.kernel(
        _gather_body,
        out_type=[jax.ShapeDtypeStruct((E, HD), jnp.float32)] * 3,
        mesh=mesh,
        scratch_types=[
            pltpu.VMEM((CH,), jnp.int32),
            pltpu.VMEM((CH,), jnp.int32),
            pltpu.VMEM((CH, HD), jnp.float32),
            pltpu.VMEM((CH, HD), jnp.float32),
            pltpu.VMEM((CH, HD), jnp.float32),
            pltpu.SemaphoreType.DMA,
        ],
    )
    return f(ktab, qtab, vtab, src, dst)


def _edge_body(ea_ref, kg_ref, qg_ref, vg_ref, wewt_ref, webt_ref,
               bw_ref, bb_ref, awrep_ref, vblk_ref,
               we_ref, p1_ref, p2_ref):
    ea = ea_ref[:]
    kq = kg_ref[:] + qg_ref[:]
    ew = jnp.dot(ea, wewt_ref[:], preferred_element_type=jnp.float32) + bw_ref[:]
    eb = jnp.dot(ea, webt_ref[:], preferred_element_type=jnp.float32) + bb_ref[:]
    sc = kq * ew
    root = jnp.sqrt(jnp.abs(sc))
    et = jnp.where(sc >= 0.0, root, -root) + eb
    we_ref[:] = et
    et2 = jnp.dot(et, vblk_ref[:], preferred_element_type=jnp.float32)
    srep = jnp.dot(et, awrep_ref[:], preferred_element_type=jnp.float32)
    srep = jnp.clip(srep, -CLAMP, CLAMP)
    mrep = jnp.exp(srep)
    p1_ref[:] = mrep * (vg_ref[:] + et2)
    col = lax.broadcasted_iota(jnp.int32, mrep.shape, 1)
    p2_ref[:] = jnp.where(col == 1, 1.0, mrep)


def _edge(edge_attr, kg, qg, vg, wewt, webt, bw, bb, awrep, vblk):
    be_blk = 4000
    grid = E // be_blk
    full = lambda i: (0, 0)
    blk = lambda i: (i, 0)
    return pl.pallas_call(
        _edge_body,
        grid=(grid,),
        in_specs=[
            pl.BlockSpec((be_blk, IN_DIM), blk),
            pl.BlockSpec((be_blk, HD), blk),
            pl.BlockSpec((be_blk, HD), blk),
            pl.BlockSpec((be_blk, HD), blk),
            pl.BlockSpec((IN_DIM, HD), full),
            pl.BlockSpec((IN_DIM, HD), full),
            pl.BlockSpec((1, HD), full),
            pl.BlockSpec((1, HD), full),
            pl.BlockSpec((HD, HD), full),
            pl.BlockSpec((HD, HD), full),
        ],
        out_specs=[
            pl.BlockSpec((be_blk, HD), blk),
            pl.BlockSpec((be_blk, HD), blk),
            pl.BlockSpec((be_blk, HD), blk),
        ],
        out_shape=[
            jax.ShapeDtypeStruct((E, HD), jnp.float32),
            jax.ShapeDtypeStruct((E, HD), jnp.float32),
            jax.ShapeDtypeStruct((E, HD), jnp.float32),
        ],
    )(edge_attr, kg, qg, vg, wewt, webt, bw, bb, awrep, vblk)


def _scatter_body(p1, p2, dst, z1, o1, o2, acc, idxb, b1, sem):
    c = lax.axis_index("c")
    s = lax.axis_index("s")
    row0 = pl.multiple_of(s * NST, NST)
    pltpu.sync_copy(z1, b1)

    def zcp(r, _):
        ro = pl.multiple_of(row0 + r * CH, CH)
        pltpu.sync_copy(b1, acc.at[pl.ds(ro, CH)])
        return 0

    lax.fori_loop(0, NST // CH, zcp, 0)
    plsc.subcore_barrier()

    def body(t, _):
        g = s + NSUB * t
        off = pl.multiple_of(g * CH, CH)
        l1 = pltpu.async_copy(dst.at[pl.ds(off, CH)], idxb, sem)
        l1.wait()

        @pl.when(c == 0)
        def _():
            pltpu.async_copy(p1.at[pl.ds(off, CH)], b1, sem).wait()

        @pl.when(c == 1)
        def _():
            pltpu.async_copy(p2.at[pl.ds(off, CH)], b1, sem).wait()

        pltpu.sync_copy(b1, acc.at[idxb], add=True)
        return 0

    trip = (NCHUNK - s + NSUB - 1) // NSUB
    lax.fori_loop(0, trip, body, 0)
    plsc.subcore_barrier()

    def wcp(r, _):
        ro = pl.multiple_of(row0 + r * CH, CH)
        pltpu.sync_copy(acc.at[pl.ds(ro, CH)], b1)

        @pl.when(c == 0)
        def _():
            pltpu.async_copy(b1, o1.at[pl.ds(ro, CH)], sem).wait()

        @pl.when(c == 1)
        def _():
            pltpu.async_copy(b1, o2.at[pl.ds(ro, CH)], sem).wait()

        return 0

    lax.fori_loop(0, NST // CH, wcp, 0)


def _scatter(p1, p2, dst):
    mesh = plsc.VectorSubcoreMesh(core_axis_name="c", subcore_axis_name="s")
    f = pl.kernel(
        _scatter_body,
        out_type=[
            jax.ShapeDtypeStruct((NP, HD), jnp.float32),
            jax.ShapeDtypeStruct((NP, HD), jnp.float32),
        ],
        mesh=mesh,
        scratch_types=[
            pltpu.VMEM_SHARED((NP, HD), jnp.float32),
            pltpu.VMEM((CH,), jnp.int32),
            pltpu.VMEM((CH, HD), jnp.float32),
            pltpu.SemaphoreType.DMA,
        ],
    )
    z1 = jnp.zeros((CH, HD), jnp.float32)
    return f(p1, p2, dst, z1)


def _node_body(m_ref, aux_ref, sel_ref, degsel_ref,
               p0_ref, p1_ref, p2_ref, p3_ref, pb_ref, out_ref):
    m = m_ref[:]
    aux = aux_ref[:]
    srep = jnp.dot(aux, sel_ref[:], preferred_element_type=jnp.float32)
    degrep = jnp.dot(aux, degsel_ref[:], preferred_element_type=jnp.float32)
    ld = jnp.log1p(degrep)
    wv = m / (srep + 1e-16)
    acc = jnp.dot(wv, p0_ref[:], preferred_element_type=jnp.float32)
    acc = acc + ld * jnp.dot(wv, p1_ref[:], preferred_element_type=jnp.float32)
    acc = acc + jnp.dot(wv, p2_ref[:], preferred_element_type=jnp.float32) / (1.0 + ld)
    acc = acc + (1.0 + 0.5 * ld) * jnp.dot(wv, p3_ref[:], preferred_element_type=jnp.float32)
    out_ref[:] = acc + pb_ref[:]


def _node(m, aux, sel, degsel, p0, p1, p2, p3, pb):
    bn = 2000
    grid = N // bn
    full = lambda i: (0, 0)
    blk = lambda i: (i, 0)
    return pl.pallas_call(
        _node_body,
        grid=(grid,),
        in_specs=[
            pl.BlockSpec((bn, HD), blk),
            pl.BlockSpec((bn, HD), blk),
            pl.BlockSpec((HD, HD), full),
            pl.BlockSpec((HD, HD), full),
            pl.BlockSpec((HD, HD), full),
            pl.BlockSpec((HD, HD), full),
            pl.BlockSpec((HD, HD), full),
            pl.BlockSpec((HD, HD), full),
            pl.BlockSpec((1, HD), full),
        ],
        out_specs=pl.BlockSpec((bn, HD), blk),
        out_shape=jax.ShapeDtypeStruct((N, HD), jnp.float32),
    )(m, aux, sel, degsel, p0, p1, p2, p3, pb)


_SEL1 = np.zeros((HD, HD), np.float32)
for _h in range(H):
    _SEL1[_h * D, _h * D:(_h + 1) * D] = 1.0
_DEGSEL = np.zeros((HD, HD), np.float32)
_DEGSEL[1, :] = 1.0


def _blockdiag(blocks):
    out = jnp.zeros((HD, HD), jnp.float32)
    for h, b in enumerate(blocks):
        out = out.at[h * D:(h + 1) * D, h * D:(h + 1) * D].set(b)
    return out


@jax.jit
def kernel(x, edge_index, edge_attr, Wq, bq, Wk, bk, Wv, bv, We, be, Aw,
           VeRow, projW, projb):
    src = edge_index[0].astype(jnp.int32)
    dst = edge_index[1].astype(jnp.int32)

    w3 = jnp.concatenate([Wq.T, Wk.T, Wv.T], axis=1)
    b3 = jnp.concatenate([bq, bk, bv]).reshape(1, 3 * HD)

    we4 = We.reshape(H, 2 * D, IN_DIM)
    wewt = we4[:, :D, :].reshape(HD, IN_DIM).T
    webt = we4[:, D:, :].reshape(HD, IN_DIM).T
    be2 = be.reshape(H, 2 * D)
    bw = be2[:, :D].reshape(1, HD)
    bb = be2[:, D:].reshape(1, HD)

    aw2 = Aw[:, :, 0]
    awrep = _blockdiag([jnp.outer(aw2[:, h], jnp.ones((D,), jnp.float32))
                        for h in range(H)])
    vblk = _blockdiag([VeRow[:, h, :] for h in range(H)])
    pjt = [_blockdiag([projW[:, j * D:(j + 1) * D].T] * H) for j in range(4)]
    pb = jnp.tile(projb, H).reshape(1, HD)

    q, k, v = _qkv(x, w3, b3)
    kg, qg, vg = _gather(k, q, v, src, dst)
    wE, pay1, pay2 = _edge(edge_attr, kg, qg, vg, wewt, webt, bw, bb,
                           awrep, vblk)
    o1, o2 = _scatter(pay1, pay2, dst)
    out = _node(o1[:N], o2[:N], jnp.asarray(_SEL1), jnp.asarray(_DEGSEL),
                pjt[0], pjt[1], pjt[2], pjt[3], pb)
    return out.reshape(N, H, D), wE


# speedup vs baseline: 69.8950x; 1.0606x over previous
import functools
import numpy as np
import jax
import jax.numpy as jnp
from jax import lax
from jax.experimental import pallas as pl
from jax.experimental.pallas import tpu as pltpu
from jax.experimental.pallas import tpu_sc as plsc

N = 10000
E = 320000
IN_DIM = 128
H = 8
D = 16
HD = H * D
CLAMP = 5.0

NC = 2
NSUB = 16
NW = NC * NSUB
CH = 128
NCHUNK = E // CH
NP = 10240
NST = NP // NSUB
NPA = NP // 8
NSTA = NPA // NSUB


def _qkv_body(x_ref, w_ref, b_ref, q_ref, k_ref, v_ref):
    out = jnp.dot(x_ref[:], w_ref[:], preferred_element_type=jnp.float32)
    out = out + b_ref[:]
    q_ref[:] = out[:, 0:HD]
    k_ref[:] = out[:, HD:2 * HD]
    v_ref[:] = out[:, 2 * HD:3 * HD]


def _qkv(x, w3, b3):
    bn = 2000
    grid = N // bn
    return pl.pallas_call(
        _qkv_body,
        grid=(grid,),
        in_specs=[
            pl.BlockSpec((bn, IN_DIM), lambda i: (i, 0)),
            pl.BlockSpec((IN_DIM, 3 * HD), lambda i: (0, 0)),
            pl.BlockSpec((1, 3 * HD), lambda i: (0, 0)),
        ],
        out_specs=[
            pl.BlockSpec((bn, HD), lambda i: (i, 0)),
            pl.BlockSpec((bn, HD), lambda i: (i, 0)),
            pl.BlockSpec((bn, HD), lambda i: (i, 0)),
        ],
        out_shape=[jax.ShapeDtypeStruct((N, HD), jnp.float32)] * 3,
    )(x, w3, b3)


def _gather_body(ktab, qtab, vtab, src, dst, kg, qg, vg,
                 sidx, didx, kbuf, qbuf, vbuf, sem):
    c = lax.axis_index("c")
    s = lax.axis_index("s")
    wid = s * NC + c

    def body(t, _):
        g = wid + NW * t
        off = pl.multiple_of(g * CH, CH)
        i1 = pltpu.async_copy(src.at[pl.ds(off, CH)], sidx, sem)
        i2 = pltpu.async_copy(dst.at[pl.ds(off, CH)], didx, sem)
        i1.wait()
        i2.wait()
        g1 = pltpu.async_copy(ktab.at[sidx], kbuf, sem)
        g2 = pltpu.async_copy(qtab.at[didx], qbuf, sem)
        g3 = pltpu.async_copy(vtab.at[sidx], vbuf, sem)
        g1.wait()
        g2.wait()
        g3.wait()
        w1 = pltpu.async_copy(kbuf, kg.at[pl.ds(off, CH)], sem)
        w2 = pltpu.async_copy(qbuf, qg.at[pl.ds(off, CH)], sem)
        w3 = pltpu.async_copy(vbuf, vg.at[pl.ds(off, CH)], sem)
        w1.wait()
        w2.wait()
        w3.wait()
        return 0

    trip = (NCHUNK - wid + NW - 1) // NW
    lax.fori_loop(0, trip, body, 0)


def _gather(ktab, qtab, vtab, src, dst):
    mesh = plsc.VectorSubcoreMesh(core_axis_name="c", subcore_axis_name="s")
    f = pl.kernel(
        _gather_body,
        out_type=[jax.ShapeDtypeStruct((E, HD), jnp.float32)] * 3,
        mesh=mesh,
        scratch_types=[
            pltpu.VMEM((CH,), jnp.int32),
            pltpu.VMEM((CH,), jnp.int32),
            pltpu.VMEM((CH, HD), jnp.float32),
            pltpu.VMEM((CH, HD), jnp.float32),
            pltpu.VMEM((CH, HD), jnp.float32),
            pltpu.SemaphoreType.DMA,
        ],
    )
    return f(ktab, qtab, vtab, src, dst)


def _edge_body(ea_ref, kg_ref, qg_ref, vg_ref, wewt_ref, webt_ref,
               bw_ref, bb_ref, awrep_ref, vblk_ref,
               we_ref, p1_ref, p2_ref):
    ea = ea_ref[:]
    kq = kg_ref[:] + qg_ref[:]
    ew = jnp.dot(ea, wewt_ref[:], preferred_element_type=jnp.float32) + bw_ref[:]
    eb = jnp.dot(ea, webt_ref[:], preferred_element_type=jnp.float32) + bb_ref[:]
    sc = kq * ew
    root = jnp.sqrt(jnp.abs(sc))
    et = jnp.where(sc >= 0.0, root, -root) + eb
    we_ref[:] = et
    et2 = jnp.dot(et, vblk_ref[:], preferred_element_type=jnp.float32)
    srep = jnp.dot(et, awrep_ref[:], preferred_element_type=jnp.float32)
    srep = jnp.clip(srep, -CLAMP, CLAMP)
    mrep = jnp.exp(srep)
    p1_ref[:] = mrep * (vg_ref[:] + et2)
    col = lax.broadcasted_iota(jnp.int32, mrep.shape, 1)
    p2_ref[:] = jnp.where(col == 1, 1.0, mrep)


def _edge(edge_attr, kg, qg, vg, wewt, webt, bw, bb, awrep, vblk):
    be_blk = 4000
    grid = E // be_blk
    full = lambda i: (0, 0)
    blk = lambda i: (i, 0)
    return pl.pallas_call(
        _edge_body,
        grid=(grid,),
        in_specs=[
            pl.BlockSpec((be_blk, IN_DIM), blk),
            pl.BlockSpec((be_blk, HD), blk),
            pl.BlockSpec((be_blk, HD), blk),
            pl.BlockSpec((be_blk, HD), blk),
            pl.BlockSpec((IN_DIM, HD), full),
            pl.BlockSpec((IN_DIM, HD), full),
            pl.BlockSpec((1, HD), full),
            pl.BlockSpec((1, HD), full),
            pl.BlockSpec((HD, HD), full),
            pl.BlockSpec((HD, HD), full),
        ],
        out_specs=[
            pl.BlockSpec((be_blk, HD), blk),
            pl.BlockSpec((be_blk, HD), blk),
            pl.BlockSpec((be_blk, HD), blk),
        ],
        out_shape=[
            jax.ShapeDtypeStruct((E, HD), jnp.float32),
            jax.ShapeDtypeStruct((E, HD), jnp.float32),
            jax.ShapeDtypeStruct((E, HD), jnp.float32),
        ],
    )(edge_attr, kg, qg, vg, wewt, webt, bw, bb, awrep, vblk)


def _scatter_body(p1, p2, dst, z1, o1, o2, acc, idxb, b1, sem):
    c = lax.axis_index("c")
    s = lax.axis_index("s")
    row0 = pl.multiple_of(s * NST, NST)
    pltpu.sync_copy(z1, b1)

    def zcp(r, _):
        ro = pl.multiple_of(row0 + r * CH, CH)
        pltpu.sync_copy(b1, acc.at[pl.ds(ro, CH)])
        return 0

    lax.fori_loop(0, NST // CH, zcp, 0)
    plsc.subcore_barrier()

    def body(t, _):
        g = s + NSUB * t
        off = pl.multiple_of(g * CH, CH)
        l1 = pltpu.async_copy(dst.at[pl.ds(off, CH)], idxb, sem)

        @pl.when(c == 0)
        def _():
            pltpu.async_copy(p1.at[pl.ds(off, CH)], b1, sem).wait()

        @pl.when(c == 1)
        def _():
            pltpu.async_copy(p2.at[pl.ds(off, CH)], b1, sem).wait()

        l1.wait()
        pltpu.sync_copy(b1, acc.at[idxb], add=True)
        return 0

    trip = (NCHUNK - s + NSUB - 1) // NSUB
    lax.fori_loop(0, trip, body, 0)
    plsc.subcore_barrier()

    def wcp(r, _):
        ro = pl.multiple_of(row0 + r * CH, CH)
        pltpu.sync_copy(acc.at[pl.ds(ro, CH)], b1)

        @pl.when(c == 0)
        def _():
            pltpu.async_copy(b1, o1.at[pl.ds(ro, CH)], sem).wait()

        @pl.when(c == 1)
        def _():
            pltpu.async_copy(b1, o2.at[pl.ds(ro, CH)], sem).wait()

        return 0

    lax.fori_loop(0, NST // CH, wcp, 0)


def _scatter(p1, p2, dst):
    mesh = plsc.VectorSubcoreMesh(core_axis_name="c", subcore_axis_name="s")
    f = pl.kernel(
        _scatter_body,
        out_type=[
            jax.ShapeDtypeStruct((NP, HD), jnp.float32),
            jax.ShapeDtypeStruct((NP, HD), jnp.float32),
        ],
        mesh=mesh,
        scratch_types=[
            pltpu.VMEM_SHARED((NP, HD), jnp.float32),
            pltpu.VMEM((CH,), jnp.int32),
            pltpu.VMEM((CH, HD), jnp.float32),
            pltpu.SemaphoreType.DMA,
        ],
    )
    z1 = jnp.zeros((CH, HD), jnp.float32)
    return f(p1, p2, dst, z1)


def _node_body(m_ref, aux_ref, sel_ref, degsel_ref,
               p0_ref, p1_ref, p2_ref, p3_ref, pb_ref, out_ref):
    m = m_ref[:]
    aux = aux_ref[:]
    srep = jnp.dot(aux, sel_ref[:], preferred_element_type=jnp.float32)
    degrep = jnp.dot(aux, degsel_ref[:], preferred_element_type=jnp.float32)
    ld = jnp.log1p(degrep)
    wv = m / (srep + 1e-16)
    acc = jnp.dot(wv, p0_ref[:], preferred_element_type=jnp.float32)
    acc = acc + ld * jnp.dot(wv, p1_ref[:], preferred_element_type=jnp.float32)
    acc = acc + jnp.dot(wv, p2_ref[:], preferred_element_type=jnp.float32) / (1.0 + ld)
    acc = acc + (1.0 + 0.5 * ld) * jnp.dot(wv, p3_ref[:], preferred_element_type=jnp.float32)
    out_ref[:] = acc + pb_ref[:]


def _node(m, aux, sel, degsel, p0, p1, p2, p3, pb):
    bn = 2000
    grid = N // bn
    full = lambda i: (0, 0)
    blk = lambda i: (i, 0)
    return pl.pallas_call(
        _node_body,
        grid=(grid,),
        in_specs=[
            pl.BlockSpec((bn, HD), blk),
            pl.BlockSpec((bn, HD), blk),
            pl.BlockSpec((HD, HD), full),
            pl.BlockSpec((HD, HD), full),
            pl.BlockSpec((HD, HD), full),
            pl.BlockSpec((HD, HD), full),
            pl.BlockSpec((HD, HD), full),
            pl.BlockSpec((HD, HD), full),
            pl.BlockSpec((1, HD), full),
        ],
        out_specs=pl.BlockSpec((bn, HD), blk),
        out_shape=jax.ShapeDtypeStruct((N, HD), jnp.float32),
    )(m, aux, sel, degsel, p0, p1, p2, p3, pb)


_SEL1 = np.zeros((HD, HD), np.float32)
for _h in range(H):
    _SEL1[_h * D, _h * D:(_h + 1) * D] = 1.0
_DEGSEL = np.zeros((HD, HD), np.float32)
_DEGSEL[1, :] = 1.0


def _blockdiag(blocks):
    out = jnp.zeros((HD, HD), jnp.float32)
    for h, b in enumerate(blocks):
        out = out.at[h * D:(h + 1) * D, h * D:(h + 1) * D].set(b)
    return out


@jax.jit
def kernel(x, edge_index, edge_attr, Wq, bq, Wk, bk, Wv, bv, We, be, Aw,
           VeRow, projW, projb):
    src = edge_index[0].astype(jnp.int32)
    dst = edge_index[1].astype(jnp.int32)

    w3 = jnp.concatenate([Wq.T, Wk.T, Wv.T], axis=1)
    b3 = jnp.concatenate([bq, bk, bv]).reshape(1, 3 * HD)

    we4 = We.reshape(H, 2 * D, IN_DIM)
    wewt = we4[:, :D, :].reshape(HD, IN_DIM).T
    webt = we4[:, D:, :].reshape(HD, IN_DIM).T
    be2 = be.reshape(H, 2 * D)
    bw = be2[:, :D].reshape(1, HD)
    bb = be2[:, D:].reshape(1, HD)

    aw2 = Aw[:, :, 0]
    awrep = _blockdiag([jnp.outer(aw2[:, h], jnp.ones((D,), jnp.float32))
                        for h in range(H)])
    vblk = _blockdiag([VeRow[:, h, :] for h in range(H)])
    pjt = [_blockdiag([projW[:, j * D:(j + 1) * D].T] * H) for j in range(4)]
    pb = jnp.tile(projb, H).reshape(1, HD)

    q, k, v = _qkv(x, w3, b3)
    kg, qg, vg = _gather(k, q, v, src, dst)
    wE, pay1, pay2 = _edge(edge_attr, kg, qg, vg, wewt, webt, bw, bb,
                           awrep, vblk)
    o1, o2 = _scatter(pay1, pay2, dst)
    out = _node(o1[:N], o2[:N], jnp.asarray(_SEL1), jnp.asarray(_DEGSEL),
                pjt[0], pjt[1], pjt[2], pjt[3], pb)
    return out.reshape(N, H, D), wE
